# Initial kernel scaffold; baseline (speedup 1.0000x reference)
import functools

import jax
import jax.numpy as jnp
from jax import lax
from jax.experimental import pallas as pl
from jax.experimental.pallas import tpu as pltpu
from jax.experimental.pallas import tpu_sc as plsc

EPS = 0.1
NC = 2
NS = 16
LANES = 16
CHUNK = 128


def _cdiv(a, b):
  return (a + b - 1) // b


def kernel(x, edge_index, W_in, b_in, att_l1, att_r1, att_l2, att_r2,
           W_cls, b_cls):
  N, D = x.shape
  E = edge_index.shape[1]
  DO = W_cls.shape[1]
  NW = NC * NS

  N_PAD = _cdiv(N + 1, 128) * 128
  STRIPE = N_PAD // NS
  CPT = _cdiv(_cdiv(E, NW), CHUNK)
  E_PAD = NW * CPT * CHUNK
  R = N_PAD // 8
  GRID = N_PAD // R

  f32 = jnp.float32

  def _proj_body(x_ref, w_ref, b_ref, al_ref, ar_ref, h_ref, hl_ref, hr_ref):
    h = jnp.dot(x_ref[...], w_ref[...], preferred_element_type=f32) + b_ref[...]
    h_ref[...] = h
    hl_ref[...] = jnp.sum(h * al_ref[...], axis=1, keepdims=True)
    hr_ref[...] = jnp.sum(h * ar_ref[...], axis=1, keepdims=True)

  _proj = pl.pallas_call(
      _proj_body,
      grid=(GRID,),
      in_specs=[
          pl.BlockSpec((R, D), lambda i: (i, 0)),
          pl.BlockSpec((D, D), lambda i: (0, 0)),
          pl.BlockSpec((1, D), lambda i: (0, 0)),
          pl.BlockSpec((1, D), lambda i: (0, 0)),
          pl.BlockSpec((1, D), lambda i: (0, 0)),
      ],
      out_specs=[
          pl.BlockSpec((R, D), lambda i: (i, 0)),
          pl.BlockSpec((R, 1), lambda i: (i, 0)),
          pl.BlockSpec((R, 1), lambda i: (i, 0)),
      ],
      out_shape=[
          jax.ShapeDtypeStruct((N_PAD, D), f32),
          jax.ShapeDtypeStruct((N_PAD, 1), f32),
          jax.ShapeDtypeStruct((N_PAD, 1), f32),
      ],
  )

  mesh = plsc.VectorSubcoreMesh(core_axis_name="c", subcore_axis_name="s")
  ZREGS = _cdiv(STRIPE, LANES)

  @functools.partial(
      pl.kernel,
      out_type=jax.ShapeDtypeStruct((NC, N_PAD), f32),
      mesh=mesh,
      scratch_types=[
          pltpu.VMEM((CPT, CHUNK), jnp.int32),
          pltpu.VMEM((CHUNK,), f32),
          pltpu.VMEM((ZREGS * LANES,), f32),
          pltpu.VMEM_SHARED((N_PAD,), f32),
      ],
  )
  def _deg(dst_hbm, out_hbm, dst_v, ones_v, zbuf, deg_sh):
    c = lax.axis_index("c")
    s = lax.axis_index("s")
    wid = s * NC + c
    pltpu.sync_copy(dst_hbm.at[pl.ds(wid * CPT, CPT)], dst_v)
    for k in range(CHUNK // LANES):
      ones_v[pl.ds(k * LANES, LANES)] = jnp.full((LANES,), 1.0, f32)
    for k in range(ZREGS):
      zbuf[pl.ds(k * LANES, LANES)] = jnp.zeros((LANES,), f32)
    pltpu.sync_copy(zbuf.at[pl.ds(0, STRIPE)],
                    deg_sh.at[pl.ds(s * STRIPE, STRIPE)])
    plsc.subcore_barrier()

    def body(j, carry):
      pltpu.sync_copy(ones_v, deg_sh.at[dst_v.at[j]], add=True)
      return carry

    lax.fori_loop(0, CPT, body, 0)
    plsc.subcore_barrier()
    pltpu.sync_copy(deg_sh.at[pl.ds(s * STRIPE, STRIPE)],
                    out_hbm.at[c, pl.ds(s * STRIPE, STRIPE)])

  NROW = N_PAD // 128

  def _dinv_body(dp_ref, dinv_ref):
    deg = dp_ref[0] + dp_ref[1]
    flat = (lax.broadcasted_iota(jnp.int32, (NROW, 128), 0) * 128 +
            lax.broadcasted_iota(jnp.int32, (NROW, 128), 1))
    valid = (flat < N) & (deg > 0)
    dinv_ref[...] = jnp.where(valid, lax.rsqrt(jnp.maximum(deg, 1e-12)), 0.0)

  _dinv = pl.pallas_call(
      _dinv_body,
      out_shape=jax.ShapeDtypeStruct((NROW, 128), f32),
  )

  @functools.partial(
      pl.kernel,
      out_type=jax.ShapeDtypeStruct((NC, N_PAD, D), f32),
      mesh=mesh,
      scratch_types=[
          pltpu.VMEM((CPT, CHUNK), jnp.int32),
          pltpu.VMEM((CPT, CHUNK), jnp.int32),
          pltpu.VMEM((N_PAD,), f32),
          pltpu.VMEM((N_PAD,), f32),
          pltpu.VMEM((N_PAD,), f32),
          pltpu.VMEM((CHUNK, D), f32),
          pltpu.VMEM((CHUNK,), f32),
          pltpu.VMEM_SHARED((N_PAD, D), f32),
          pltpu.SemaphoreType.DMA,
      ],
  )
  def _edge(h_hbm, hl_hbm, hr_hbm, dinv_hbm, src_hbm, dst_hbm, out_hbm,
            src_v, dst_v, hl_v, hr_v, dinv_v, rows_v, coef_v, acc_sh, sem):
    c = lax.axis_index("c")
    s = lax.axis_index("s")
    wid = s * NC + c
    pltpu.sync_copy(src_hbm.at[pl.ds(wid * CPT, CPT)], src_v)
    pltpu.sync_copy(dst_hbm.at[pl.ds(wid * CPT, CPT)], dst_v)
    pltpu.sync_copy(hl_hbm, hl_v)
    pltpu.sync_copy(hr_hbm, hr_v)
    pltpu.sync_copy(dinv_hbm, dinv_v)

    def zrow(i, carry):
      for k in range(D // LANES):
        rows_v[i, pl.ds(k * LANES, LANES)] = jnp.zeros((LANES,), f32)
      return carry

    lax.fori_loop(0, CHUNK, zrow, 0)
    base = s * STRIPE
    off = 0
    while off < STRIPE:
      nrows = min(CHUNK, STRIPE - off)
      pltpu.sync_copy(rows_v.at[pl.ds(0, nrows)],
                      acc_sh.at[pl.ds(base + off, nrows)])
      off += nrows
    plsc.subcore_barrier()

    def chunk(j, carry):
      cp = pltpu.async_copy(h_hbm.at[src_v.at[j]], rows_v, sem)
      for k in range(CHUNK // LANES):
        sv = src_v[j, pl.ds(k * LANES, LANES)]
        dv = dst_v[j, pl.ds(k * LANES, LANES)]
        a = plsc.load_gather(hl_v, [sv]) + plsc.load_gather(hr_v, [dv])
        u = jnp.exp(-2.0 * jnp.abs(a))
        r = (1.0 - u) / (1.0 + u)
        th = jnp.where(a < 0, -r, r)
        coef_v[pl.ds(k * LANES, LANES)] = th * plsc.load_gather(dinv_v, [sv])
      cp.wait()

      def scale(i, cc):
        aco = coef_v[i]
        for k in range(D // LANES):
          rows_v[i, pl.ds(k * LANES, LANES)] = (
              rows_v[i, pl.ds(k * LANES, LANES)] * aco)
        return cc

      lax.fori_loop(0, CHUNK, scale, 0)
      pltpu.sync_copy(rows_v, acc_sh.at[dst_v.at[j]], add=True)
      return carry

    lax.fori_loop(0, CPT, chunk, 0)
    plsc.subcore_barrier()
    pltpu.sync_copy(acc_sh.at[pl.ds(base, STRIPE)],
                    out_hbm.at[c, pl.ds(base, STRIPE)])

  def _comb_body(p_ref, x0_ref, dinv_ref, al_ref, ar_ref,
                 hout_ref, hl_ref, hr_ref):
    acc = p_ref[0] + p_ref[1]
    h1 = dinv_ref[...] * acc + EPS * x0_ref[...]
    hr_ = jnp.maximum(h1, 0.0)
    hout_ref[...] = hr_
    hl_ref[...] = jnp.sum(hr_ * al_ref[...], axis=1, keepdims=True)
    hr_ref[...] = jnp.sum(hr_ * ar_ref[...], axis=1, keepdims=True)

  _comb = pl.pallas_call(
      _comb_body,
      grid=(GRID,),
      in_specs=[
          pl.BlockSpec((2, R, D), lambda i: (0, i, 0)),
          pl.BlockSpec((R, D), lambda i: (i, 0)),
          pl.BlockSpec((R, 1), lambda i: (i, 0)),
          pl.BlockSpec((1, D), lambda i: (0, 0)),
          pl.BlockSpec((1, D), lambda i: (0, 0)),
      ],
      out_specs=[
          pl.BlockSpec((R, D), lambda i: (i, 0)),
          pl.BlockSpec((R, 1), lambda i: (i, 0)),
          pl.BlockSpec((R, 1), lambda i: (i, 0)),
      ],
      out_shape=[
          jax.ShapeDtypeStruct((N_PAD, D), f32),
          jax.ShapeDtypeStruct((N_PAD, 1), f32),
          jax.ShapeDtypeStruct((N_PAD, 1), f32),
      ],
  )

  def _final_body(p_ref, x0_ref, dinv_ref, wc_ref, bc_ref,
                  lg_ref, emb_ref, soft_ref, hard_ref):
    acc = p_ref[0] + p_ref[1]
    h2 = dinv_ref[...] * acc + EPS * x0_ref[...]
    emb_ref[...] = h2
    lg = jnp.dot(h2, wc_ref[...], preferred_element_type=f32) + bc_ref[...]
    lg_ref[...] = lg
    m = jnp.max(lg, axis=1, keepdims=True)
    e = jnp.exp(lg - m)
    soft = e / jnp.sum(e, axis=1, keepdims=True)
    soft_ref[...] = soft
    colidx = lax.broadcasted_iota(jnp.int32, (R, DO), 1)
    ismax = soft == jnp.max(soft, axis=1, keepdims=True)
    hard_ref[...] = jnp.min(jnp.where(ismax, colidx, DO), axis=1,
                            keepdims=True)

  _final = pl.pallas_call(
      _final_body,
      grid=(GRID,),
      in_specs=[
          pl.BlockSpec((2, R, D), lambda i: (0, i, 0)),
          pl.BlockSpec((R, D), lambda i: (i, 0)),
          pl.BlockSpec((R, 1), lambda i: (i, 0)),
          pl.BlockSpec((D, DO), lambda i: (0, 0)),
          pl.BlockSpec((1, DO), lambda i: (0, 0)),
      ],
      out_specs=[
          pl.BlockSpec((R, DO), lambda i: (i, 0)),
          pl.BlockSpec((R, D), lambda i: (i, 0)),
          pl.BlockSpec((R, DO), lambda i: (i, 0)),
          pl.BlockSpec((R, 1), lambda i: (i, 0)),
      ],
      out_shape=[
          jax.ShapeDtypeStruct((N_PAD, DO), f32),
          jax.ShapeDtypeStruct((N_PAD, D), f32),
          jax.ShapeDtypeStruct((N_PAD, DO), f32),
          jax.ShapeDtypeStruct((N_PAD, 1), jnp.int32),
      ],
  )

  x_p = jnp.pad(x, ((0, N_PAD - N), (0, 0)))
  src2d = jnp.pad(edge_index[0], (0, E_PAD - E),
                  constant_values=N).reshape(NW * CPT, CHUNK)
  dst2d = jnp.pad(edge_index[1], (0, E_PAD - E),
                  constant_values=N).reshape(NW * CPT, CHUNK)

  h_p, hl1, hr1 = _proj(x_p, W_in, b_in.reshape(1, D),
                        att_l1.reshape(1, D), att_r1.reshape(1, D))
  deg_parts = _deg(dst2d)
  dinv2d = _dinv(deg_parts.reshape(NC, NROW, 128))
  dinv_flat = dinv2d.reshape(N_PAD)
  dinv_col = dinv2d.reshape(N_PAD, 1)

  parts1 = _edge(h_p, hl1.reshape(N_PAD), hr1.reshape(N_PAD), dinv_flat,
                 src2d, dst2d)
  hrelu, hl2, hr2 = _comb(parts1, h_p, dinv_col,
                          att_l2.reshape(1, D), att_r2.reshape(1, D))
  parts2 = _edge(hrelu, hl2.reshape(N_PAD), hr2.reshape(N_PAD), dinv_flat,
                 src2d, dst2d)
  logits_p, emb_p, soft_p, hard_p = _final(parts2, h_p, dinv_col,
                                           W_cls, b_cls.reshape(1, DO))
  return (logits_p[:N], emb_p[:N], soft_p[:N], hard_p[:N, 0])


# speedup vs baseline: 15.4956x; 15.4956x over previous
import functools

import jax
import jax.numpy as jnp
from jax import lax
from jax.experimental import pallas as pl
from jax.experimental.pallas import tpu as pltpu
from jax.experimental.pallas import tpu_sc as plsc

EPS = 0.1
NC = 2
NS = 16
LANES = 16
KCH = 64
WIN = 8
DCH = 128


def _cdiv(a, b):
  return (a + b - 1) // b


def kernel(x, edge_index, W_in, b_in, att_l1, att_r1, att_l2, att_r2,
           W_cls, b_cls):
  N, D = x.shape
  E = edge_index.shape[1]
  DO = W_cls.shape[1]
  NW = NC * NS

  N_PAD = _cdiv(N + 1, 128) * 128
  STRIPE = N_PAD // NS
  NROW = N_PAD // 128
  NCH = _cdiv(_cdiv(E, NW), KCH * WIN) * WIN
  EPT = NCH * KCH
  E_PAD = NW * EPT
  CPT = EPT // DCH
  NWIN = NCH // WIN
  R = N_PAD // 8
  GRID = N_PAD // R

  f32 = jnp.float32

  def _proj_body(x_ref, w_ref, b_ref, al_ref, ar_ref, dinv_ref,
                 h_ref, hp_ref, hl_ref, hr_ref):
    h = jnp.dot(x_ref[...], w_ref[...], preferred_element_type=f32) + b_ref[...]
    h_ref[...] = h
    hp_ref[...] = dinv_ref[...] * h
    hl_ref[...] = jnp.tanh(jnp.dot(h, al_ref[...], preferred_element_type=f32))
    hr_ref[...] = jnp.tanh(jnp.dot(h, ar_ref[...], preferred_element_type=f32))

  _proj = pl.pallas_call(
      _proj_body,
      grid=(GRID,),
      in_specs=[
          pl.BlockSpec((R, D), lambda i: (i, 0)),
          pl.BlockSpec((D, D), lambda i: (0, 0)),
          pl.BlockSpec((1, D), lambda i: (0, 0)),
          pl.BlockSpec((D, 1), lambda i: (0, 0)),
          pl.BlockSpec((D, 1), lambda i: (0, 0)),
          pl.BlockSpec((R, 1), lambda i: (i, 0)),
      ],
      out_specs=[
          pl.BlockSpec((R, D), lambda i: (i, 0)),
          pl.BlockSpec((R, D), lambda i: (i, 0)),
          pl.BlockSpec((R, 1), lambda i: (i, 0)),
          pl.BlockSpec((R, 1), lambda i: (i, 0)),
      ],
      out_shape=[
          jax.ShapeDtypeStruct((N_PAD, D), f32),
          jax.ShapeDtypeStruct((N_PAD, D), f32),
          jax.ShapeDtypeStruct((N_PAD, 1), f32),
          jax.ShapeDtypeStruct((N_PAD, 1), f32),
      ],
  )

  mesh = plsc.VectorSubcoreMesh(core_axis_name="c", subcore_axis_name="s",
                                num_cores=NC, num_subcores=NS)
  sc_params = pltpu.CompilerParams(use_tc_tiling_on_sc=False,
                                   needs_layout_passes=False)
  ZREGS = _cdiv(STRIPE, LANES)

  @functools.partial(
      pl.kernel,
      out_type=jax.ShapeDtypeStruct((NC * N_PAD,), f32),
      mesh=mesh,
      compiler_params=sc_params,
      scratch_types=[
          pltpu.VMEM((CPT, DCH), jnp.int32),
          pltpu.VMEM((DCH,), f32),
          pltpu.VMEM((ZREGS * LANES,), f32),
          pltpu.VMEM_SHARED((N_PAD,), f32),
      ],
  )
  def _deg(dst_hbm, out_hbm, dst_v, ones_v, zbuf, deg_sh):
    c = lax.axis_index("c")
    s = lax.axis_index("s")
    wid = s * NC + c
    pltpu.sync_copy(dst_hbm.at[wid], dst_v)
    for k in range(DCH // LANES):
      ones_v[pl.ds(k * LANES, LANES)] = jnp.full((LANES,), 1.0, f32)
    for k in range(ZREGS):
      zbuf[pl.ds(k * LANES, LANES)] = jnp.zeros((LANES,), f32)
    pltpu.sync_copy(zbuf.at[pl.ds(0, STRIPE)],
                    deg_sh.at[pl.ds(s * STRIPE, STRIPE)])
    plsc.subcore_barrier()

    def body(j, carry):
      pltpu.sync_copy(ones_v, deg_sh.at[dst_v.at[j]], add=True)
      return carry

    lax.fori_loop(0, CPT, body, 0)
    plsc.subcore_barrier()
    pltpu.sync_copy(deg_sh.at[pl.ds(s * STRIPE, STRIPE)],
                    zbuf.at[pl.ds(0, STRIPE)])
    pltpu.sync_copy(zbuf.at[pl.ds(0, STRIPE)],
                    out_hbm.at[pl.ds(c * N_PAD + s * STRIPE, STRIPE)])

  def _dinv_body(dp_ref, dinv_ref):
    deg = dp_ref[0] + dp_ref[1]
    flat = (lax.broadcasted_iota(jnp.int32, (NROW, 128), 0) * 128 +
            lax.broadcasted_iota(jnp.int32, (NROW, 128), 1))
    valid = (flat < N) & (deg > 0)
    dinv_ref[...] = jnp.where(valid, lax.rsqrt(jnp.maximum(deg, 1e-12)), 0.0)

  _dinv = pl.pallas_call(
      _dinv_body,
      out_shape=jax.ShapeDtypeStruct((NROW, 128), f32),
  )

  @functools.partial(
      pl.kernel,
      out_type=jax.ShapeDtypeStruct((NW, NCH, KCH), f32),
      mesh=mesh,
      compiler_params=sc_params,
      scratch_types=[
          pltpu.VMEM((N_PAD,), f32),
          pltpu.VMEM((N_PAD,), f32),
          pltpu.VMEM((NCH, 2, KCH), jnp.int32),
          pltpu.VMEM((NCH, KCH), f32),
      ],
  )
  def _coef(e3_hbm, thl_hbm, thr_hbm, th_out, thl_v, thr_v, e_v, th_buf):
    c = lax.axis_index("c")
    s = lax.axis_index("s")
    wid = s * NC + c
    pltpu.sync_copy(e3_hbm.at[wid], e_v)
    pltpu.sync_copy(thl_hbm, thl_v)
    pltpu.sync_copy(thr_hbm, thr_v)

    def chunk(j, carry):
      def grp(g, cc):
        sv = e_v[j, 0, pl.ds(g * LANES, LANES)]
        dv = e_v[j, 1, pl.ds(g * LANES, LANES)]
        ta = plsc.load_gather(thl_v, [sv])
        tb = plsc.load_gather(thr_v, [dv])
        th_buf[j, pl.ds(g * LANES, LANES)] = (
            (ta + tb) / jnp.maximum(1.0 + ta * tb, 1e-30))
        return cc

      lax.fori_loop(0, KCH // LANES, grp, 0)
      return carry

    lax.fori_loop(0, NCH, chunk, 0)
    pltpu.sync_copy(th_buf, th_out.at[wid])

  NB = 4

  @functools.partial(
      pl.kernel,
      out_type=jax.ShapeDtypeStruct((NC, N_PAD, D), f32),
      mesh=mesh,
      compiler_params=sc_params,
      scratch_types=[
          pltpu.VMEM((NB * KCH, D), f32),
          pltpu.VMEM((2, WIN, 2, KCH), jnp.int32),
          pltpu.VMEM((2, WIN, KCH), f32),
          pltpu.VMEM_SHARED((N_PAD, D), f32),
          pltpu.SemaphoreType.DMA,
          pltpu.SemaphoreType.DMA,
          pltpu.SemaphoreType.DMA,
          pltpu.SemaphoreType.DMA,
          pltpu.SemaphoreType.DMA,
          pltpu.SemaphoreType.DMA,
          pltpu.SemaphoreType.DMA,
          pltpu.SemaphoreType.DMA,
          pltpu.SemaphoreType.DMA,
      ],
  )
  def _edge(hp_hbm, e3_hbm, th_hbm, out_hbm,
            rows_b, we, wt, acc_sh,
            g0, g1, g2, g3, s0, s1, s2, s3, es):
    gsem = (g0, g1, g2, g3)
    ssem = (s0, s1, s2, s3)
    c = lax.axis_index("c")
    s = lax.axis_index("s")
    wid = s * NC + c
    base = s * STRIPE

    def zrow(i, carry):
      for k in range(D // LANES):
        rows_b[i, pl.ds(k * LANES, LANES)] = jnp.zeros((LANES,), f32)
      return carry

    lax.fori_loop(0, NB * KCH, zrow, 0)
    off = 0
    while off < STRIPE:
      nrows = min(NB * KCH, STRIPE - off)
      pltpu.sync_copy(rows_b.at[pl.ds(0, nrows)],
                      acc_sh.at[pl.ds(base + off, nrows)])
      off += nrows
    plsc.subcore_barrier()

    pltpu.sync_copy(e3_hbm.at[wid, pl.ds(0, WIN)], we.at[0])
    pltpu.sync_copy(th_hbm.at[wid, pl.ds(0, WIN)], wt.at[0])
    pltpu.async_copy(hp_hbm.at[we.at[0, 0, 0]], rows_b.at[pl.ds(0, KCH)], g0)
    pltpu.async_copy(hp_hbm.at[we.at[0, 1, 0]], rows_b.at[pl.ds(KCH, KCH)], g1)
    pltpu.async_copy(rows_b.at[pl.ds(2 * KCH, KCH)],
                     acc_sh.at[we.at[0, 0, 1]], s2, add=True)
    pltpu.async_copy(rows_b.at[pl.ds(3 * KCH, KCH)],
                     acc_sh.at[we.at[0, 1, 1]], s3, add=True)

    def window(t, carry):
      u = t % 2
      un = 1 - u
      tn = jnp.minimum(t + 1, NWIN - 1)
      desc = []
      for cc in range(WIN):
        p = cc % NB
        r = (cc + 2) % NB
        pltpu.make_async_copy(hp_hbm.at[we.at[u, cc, 0]],
                              rows_b.at[pl.ds(p * KCH, KCH)], gsem[p]).wait()
        pltpu.make_async_copy(rows_b.at[pl.ds(r * KCH, KCH)],
                              acc_sh.at[we.at[u, cc, 1]], ssem[r]).wait()
        if cc == 2:
          desc.append(pltpu.async_copy(e3_hbm.at[wid, pl.ds(tn * WIN, WIN)],
                                       we.at[un], es))
          desc.append(pltpu.async_copy(th_hbm.at[wid, pl.ds(tn * WIN, WIN)],
                                       wt.at[un], es))
        if cc == WIN - 2:
          desc[0].wait()
          desc[1].wait()
        if cc < WIN - 2:
          srcref = we.at[u, cc + 2, 0]
        else:
          srcref = we.at[un, cc + 2 - WIN, 0]
        pltpu.async_copy(hp_hbm.at[srcref],
                         rows_b.at[pl.ds(r * KCH, KCH)], gsem[r])

        def grp(g, cg):
          c16 = wt[u, cc, pl.ds(g * LANES, LANES)]
          for ii in range(LANES):
            aco = c16[ii]
            row = p * KCH + g * LANES + ii
            for k in range(D // LANES):
              rows_b[row, pl.ds(k * LANES, LANES)] = (
                  rows_b[row, pl.ds(k * LANES, LANES)] * aco)
          return cg

        lax.fori_loop(0, KCH // LANES, grp, 0)
        pltpu.async_copy(rows_b.at[pl.ds(p * KCH, KCH)],
                         acc_sh.at[we.at[u, cc, 1]], ssem[p], add=True)
      return carry

    lax.fori_loop(0, NWIN, window, 0)

    ul = (NWIN - 1) % 2
    pltpu.make_async_copy(rows_b.at[pl.ds(2 * KCH, KCH)],
                          acc_sh.at[we.at[ul, WIN - 2, 1]], s2).wait()
    pltpu.make_async_copy(rows_b.at[pl.ds(3 * KCH, KCH)],
                          acc_sh.at[we.at[ul, WIN - 1, 1]], s3).wait()
    pltpu.make_async_copy(hp_hbm.at[we.at[ul, 0, 0]],
                          rows_b.at[pl.ds(0, KCH)], g0).wait()
    pltpu.make_async_copy(hp_hbm.at[we.at[ul, 1, 0]],
                          rows_b.at[pl.ds(KCH, KCH)], g1).wait()
    plsc.subcore_barrier()
    off = 0
    while off < STRIPE:
      nrows = min(NB * KCH, STRIPE - off)
      pltpu.sync_copy(acc_sh.at[pl.ds(base + off, nrows)],
                      rows_b.at[pl.ds(0, nrows)])
      pltpu.sync_copy(rows_b.at[pl.ds(0, nrows)],
                      out_hbm.at[c, pl.ds(base + off, nrows)])
      off += nrows

  def _comb_body(p_ref, x0_ref, dinv_ref, al_ref, ar_ref,
                 hp_ref, hl_ref, hr_ref):
    acc = p_ref[0] + p_ref[1]
    h1 = dinv_ref[...] * acc + EPS * x0_ref[...]
    hr_ = jnp.maximum(h1, 0.0)
    hp_ref[...] = dinv_ref[...] * hr_
    hl_ref[...] = jnp.tanh(jnp.dot(hr_, al_ref[...], preferred_element_type=f32))
    hr_ref[...] = jnp.tanh(jnp.dot(hr_, ar_ref[...], preferred_element_type=f32))

  _comb = pl.pallas_call(
      _comb_body,
      grid=(GRID,),
      in_specs=[
          pl.BlockSpec((2, R, D), lambda i: (0, i, 0)),
          pl.BlockSpec((R, D), lambda i: (i, 0)),
          pl.BlockSpec((R, 1), lambda i: (i, 0)),
          pl.BlockSpec((D, 1), lambda i: (0, 0)),
          pl.BlockSpec((D, 1), lambda i: (0, 0)),
      ],
      out_specs=[
          pl.BlockSpec((R, D), lambda i: (i, 0)),
          pl.BlockSpec((R, 1), lambda i: (i, 0)),
          pl.BlockSpec((R, 1), lambda i: (i, 0)),
      ],
      out_shape=[
          jax.ShapeDtypeStruct((N_PAD, D), f32),
          jax.ShapeDtypeStruct((N_PAD, 1), f32),
          jax.ShapeDtypeStruct((N_PAD, 1), f32),
      ],
  )

  def _final_body(p_ref, x0_ref, dinv_ref, wc_ref, bc_ref,
                  lg_ref, emb_ref, soft_ref, hard_ref):
    acc = p_ref[0] + p_ref[1]
    h2 = dinv_ref[...] * acc + EPS * x0_ref[...]
    emb_ref[...] = h2
    lg = jnp.dot(h2, wc_ref[...], preferred_element_type=f32) + bc_ref[...]
    lg_ref[...] = lg
    m = jnp.max(lg, axis=1, keepdims=True)
    e = jnp.exp(lg - m)
    soft = e / jnp.sum(e, axis=1, keepdims=True)
    soft_ref[...] = soft
    colidx = lax.broadcasted_iota(jnp.int32, (R, DO), 1)
    ismax = soft == jnp.max(soft, axis=1, keepdims=True)
    hard_ref[...] = jnp.min(jnp.where(ismax, colidx, DO), axis=1,
                            keepdims=True)

  _final = pl.pallas_call(
      _final_body,
      grid=(GRID,),
      in_specs=[
          pl.BlockSpec((2, R, D), lambda i: (0, i, 0)),
          pl.BlockSpec((R, D), lambda i: (i, 0)),
          pl.BlockSpec((R, 1), lambda i: (i, 0)),
          pl.BlockSpec((D, DO), lambda i: (0, 0)),
          pl.BlockSpec((1, DO), lambda i: (0, 0)),
      ],
      out_specs=[
          pl.BlockSpec((R, DO), lambda i: (i, 0)),
          pl.BlockSpec((R, D), lambda i: (i, 0)),
          pl.BlockSpec((R, DO), lambda i: (i, 0)),
          pl.BlockSpec((R, 1), lambda i: (i, 0)),
      ],
      out_shape=[
          jax.ShapeDtypeStruct((N_PAD, DO), f32),
          jax.ShapeDtypeStruct((N_PAD, D), f32),
          jax.ShapeDtypeStruct((N_PAD, DO), f32),
          jax.ShapeDtypeStruct((N_PAD, 1), jnp.int32),
      ],
  )

  x_p = jnp.pad(x, ((0, N_PAD - N), (0, 0)))
  src_p = jnp.pad(edge_index[0], (0, E_PAD - E), constant_values=N)
  dst_p = jnp.pad(edge_index[1], (0, E_PAD - E), constant_values=N)
  dst3d = dst_p.reshape(NW, CPT, DCH)
  e3 = jnp.stack([src_p.reshape(NW, NCH, KCH),
                  dst_p.reshape(NW, NCH, KCH)], axis=2)

  deg_parts = _deg(dst3d)
  dinv2d = _dinv(deg_parts.reshape(NC, NROW, 128))
  dinv_col = dinv2d.reshape(N_PAD, 1)

  h_p, hp1, thl1, thr1 = _proj(x_p, W_in, b_in.reshape(1, D),
                               att_l1.reshape(D, 1), att_r1.reshape(D, 1),
                               dinv_col)
  th1 = _coef(e3, thl1.reshape(N_PAD), thr1.reshape(N_PAD))
  parts1 = _edge(hp1, e3, th1)
  hp2, thl2, thr2 = _comb(parts1, h_p, dinv_col,
                          att_l2.reshape(D, 1), att_r2.reshape(D, 1))
  th2 = _coef(e3, thl2.reshape(N_PAD), thr2.reshape(N_PAD))
  parts2 = _edge(hp2, e3, th2)
  logits_p, emb_p, soft_p, hard_p = _final(parts2, h_p, dinv_col,
                                           W_cls, b_cls.reshape(1, DO))
  return (logits_p[:N], emb_p[:N], soft_p[:N], hard_p[:N, 0])
